# Initial kernel scaffold; baseline (speedup 1.0000x reference)
import jax
import jax.numpy as jnp
from jax.experimental import pallas as pl


def kernel(x, edge_index, edge_type, edge_weight, W1, b1, W2, b2, W3, b3, Wg, a_src, a_dst, eb, Wjk, bjk, Wlr1, blr1, Wlr2, blr2, Wrec, brec):
    raise NotImplementedError("write your pallas kernel here")


# speedup vs baseline: 36.2855x; 36.2855x over previous
import functools
import jax
import jax.numpy as jnp
from jax import lax
from jax.experimental import pallas as pl
from jax.experimental.pallas import tpu as pltpu
from jax.experimental.pallas import tpu_sc as plsc

N = 50000
E = 800000
G = 256
D = 64
H = 8
Dh = D // H
L = 3
NET = 2

RB = 400
N_BLOCKS = N // RB


def _enc_body(x_ref, w1_ref, b1_ref, w2_ref, b2_ref, w3_ref, b3_ref, z_ref):
    h = jnp.maximum(x_ref[...] @ w1_ref[...] + b1_ref[...], 0.0)
    h = jnp.maximum(h @ w2_ref[...] + b2_ref[...], 0.0)
    z_ref[...] = h @ w3_ref[...] + b3_ref[...]


def _encoder(x, W1, b1, W2, b2, W3, b3):
    full = lambda s: pl.BlockSpec(s, lambda i: (0, 0))
    return pl.pallas_call(
        _enc_body,
        grid=(N_BLOCKS,),
        in_specs=[
            pl.BlockSpec((RB, G), lambda i: (i, 0)),
            full((G, 512)), full((1, 512)),
            full((512, 256)), full((1, 256)),
            full((256, D)), full((1, D)),
        ],
        out_specs=pl.BlockSpec((RB, D), lambda i: (i, 0)),
        out_shape=jax.ShapeDtypeStruct((N, D), jnp.float32),
    )(x, W1, b1.reshape(1, -1), W2, b2.reshape(1, -1), W3, b3.reshape(1, -1))


def _prep_body(h_ref, wg_ref, asd_ref, hp_ref, ss_ref):
    hp = h_ref[...] @ wg_ref[...]
    hp_ref[...] = hp
    ss_ref[...] = hp @ asd_ref[...]


def _gat_prep(h, Wg_l, Asd):
    hp, ss = pl.pallas_call(
        _prep_body,
        grid=(N_BLOCKS,),
        in_specs=[
            pl.BlockSpec((RB, D), lambda i: (i, 0)),
            pl.BlockSpec((D, D), lambda i: (0, 0)),
            pl.BlockSpec((D, 2 * H), lambda i: (0, 0)),
        ],
        out_specs=[
            pl.BlockSpec((RB, D), lambda i: (i, 0)),
            pl.BlockSpec((RB, 2 * H), lambda i: (i, 0)),
        ],
        out_shape=[
            jax.ShapeDtypeStruct((N, D), jnp.float32),
            jax.ShapeDtypeStruct((N, 2 * H), jnp.float32),
        ],
    )(h, Wg_l, Asd)
    return hp, ss


def _head_body(h1_ref, h2_ref, h3_ref, wj1_ref, wj2_ref, wj3_ref, bjk_ref,
               wla_ref, wlb_ref, wrec_ref, brec_ref,
               p_ref, q_ref, rec_ref):
    zg = (h1_ref[...] @ wj1_ref[...] + h2_ref[...] @ wj2_ref[...]
          + h3_ref[...] @ wj3_ref[...] + bjk_ref[...])
    p_ref[...] = zg @ wla_ref[...]
    q_ref[...] = zg @ wlb_ref[...]
    rec_ref[...] = zg @ wrec_ref[...] + brec_ref[...]


def _jk_head(h1, h2, h3, Wjk, bjk, Wlr1, Wrec, brec):
    full = lambda s: pl.BlockSpec(s, lambda i: (0, 0))
    rb = lambda k: pl.BlockSpec((RB, k), lambda i: (i, 0))
    return pl.pallas_call(
        _head_body,
        grid=(N_BLOCKS,),
        in_specs=[rb(D), rb(D), rb(D),
                  full((D, D)), full((D, D)), full((D, D)), full((1, D)),
                  full((D, D)), full((D, D)), full((D, G)), full((1, G))],
        out_specs=[rb(D), rb(D), rb(G)],
        out_shape=[
            jax.ShapeDtypeStruct((N, D), jnp.float32),
            jax.ShapeDtypeStruct((N, D), jnp.float32),
            jax.ShapeDtypeStruct((N, G), jnp.float32),
        ],
    )(h1, h2, h3, Wjk[:D], Wjk[D:2 * D], Wjk[2 * D:], bjk.reshape(1, -1),
      Wlr1[:D], Wlr1[D:], Wrec, brec.reshape(1, -1))


CH = 128
NCHUNK = E // CH
NC, NS, NW = 2, 16, 32


def _worker_bounds(cid, sid):
    wid = sid * NC + cid
    per = NCHUNK // NW
    rem = NCHUNK - per * NW
    base = wid * per + jnp.minimum(wid, rem)
    cnt = per + jnp.where(wid < rem, 1, 0)
    return base, cnt


_sc_mesh = plsc.VectorSubcoreMesh(core_axis_name="c", subcore_axis_name="s")


@functools.partial(
    pl.kernel, mesh=_sc_mesh,
    compiler_params=pltpu.CompilerParams(needs_layout_passes=False, use_tc_tiling_on_sc=False),
    out_type=[
        jax.ShapeDtypeStruct((E, 16), jnp.float32),
        jax.ShapeDtypeStruct((NC, N, 16), jnp.float32),
    ],
    scratch_types=[
        pltpu.VMEM((CH,), jnp.int32),
        pltpu.VMEM((CH,), jnp.int32),
        pltpu.VMEM((CH,), jnp.int32),
        pltpu.VMEM((CH, 16), jnp.float32),
        pltpu.VMEM((CH, 16), jnp.float32),
        pltpu.VMEM((CH, 16), jnp.float32),
        pltpu.VMEM((2 * H,), jnp.float32),
        pltpu.VMEM_SHARED((N, 16), jnp.float32),
        pltpu.SemaphoreType.DMA,
        pltpu.SemaphoreType.DMA,
    ],
)
def _sc_pass_a(src_hbm, dst_hbm, et_hbm, ss_hbm, eb_hbm, zero_hbm,
               ex_hbm, dpart_hbm,
               src_v, dst_v, et_v, a_r, b_r, ex_r, eb_v, den_sh,
               sem0, sem1):
    cid = lax.axis_index("c")
    sid = lax.axis_index("s")
    base, cnt = _worker_bounds(cid, sid)

    @pl.when(sid == 0)
    def _():
        pltpu.sync_copy(zero_hbm, den_sh)
    pltpu.sync_copy(eb_hbm, eb_v)
    pltpu.sync_copy(zero_hbm.at[pl.ds(0, CH)], ex_r)
    plsc.subcore_barrier()

    iota = lax.iota(jnp.int32, 16)
    h_idx = lax.bitwise_and(iota, 7)
    e_base = lax.shift_right_logical(iota, 3)

    def chunk_body(c, carry):
        off = c * CH
        pltpu.sync_copy(src_hbm.at[pl.ds(off, CH)], src_v)
        pltpu.sync_copy(dst_hbm.at[pl.ds(off, CH)], dst_v)
        pltpu.sync_copy(et_hbm.at[pl.ds(off, CH)], et_v)
        pltpu.async_copy(ss_hbm.at[src_v], a_r, sem0).wait()
        pltpu.async_copy(ss_hbm.at[dst_v], b_r, sem1).wait()

        def vec_body(j, carry2):
            e_idx = e_base + 2 * j
            sv = plsc.load_gather(a_r, [e_idx, h_idx])
            dv = plsc.load_gather(b_r, [e_idx, h_idx + 8])
            etv = plsc.load_gather(et_v, [e_idx])
            ebv = plsc.load_gather(eb_v, [etv * H + h_idx])
            xv = sv + dv + ebv
            xv = jnp.maximum(xv, 0.2 * xv)
            plsc.store_scatter(ex_r, [e_idx, h_idx], jnp.exp(xv))
            return carry2

        lax.fori_loop(0, CH * H // 16, vec_body, 0)
        pltpu.sync_copy(ex_r, ex_hbm.at[pl.ds(off, CH)])
        pltpu.sync_copy(ex_r, den_sh.at[dst_v], add=True)
        return carry

    lax.fori_loop(base, base + cnt, chunk_body, 0)
    plsc.subcore_barrier()

    @pl.when(sid == 0)
    def _():
        pltpu.sync_copy(den_sh, dpart_hbm.at[cid])


def _make_sc_pass_b(hoff):
    @functools.partial(
        pl.kernel, mesh=_sc_mesh,
        compiler_params=pltpu.CompilerParams(needs_layout_passes=False, use_tc_tiling_on_sc=False),
        out_type=jax.ShapeDtypeStruct((NC, N, D // 2), jnp.float32),
        scratch_types=[
            pltpu.VMEM((CH,), jnp.int32),
            pltpu.VMEM((CH,), jnp.int32),
            pltpu.VMEM((CH,), jnp.float32),
            pltpu.VMEM((CH, D // 2), jnp.float32),
            pltpu.VMEM((CH, 16), jnp.float32),
            pltpu.VMEM((CH, 16), jnp.float32),
            pltpu.VMEM((CH * H // 2,), jnp.float32),
            pltpu.VMEM((CH, D // 2), jnp.float32),
            pltpu.VMEM_SHARED((N, D // 2), jnp.float32),
            pltpu.SemaphoreType.DMA,
            pltpu.SemaphoreType.DMA,
        ],
    )
    def _sc_pass_b(src_hbm, dst_hbm, ew_hbm, hp_hbm, ex_hbm, den_hbm, zero_hbm,
                   out_hbm,
                   src_v, dst_v, ew_v, hp_r, ex_r, dn_r, w_r, msg_r, out_sh,
                   sem0, sem1):
        cid = lax.axis_index("c")
        sid = lax.axis_index("s")
        base, cnt = _worker_bounds(cid, sid)

        @pl.when(sid == 0)
        def _():
            pltpu.sync_copy(zero_hbm, out_sh)
        plsc.subcore_barrier()

        iota = lax.iota(jnp.int32, 16)
        hl_idx = lax.bitwise_and(iota, 3)
        e_base4 = lax.shift_right_logical(iota, 2)
        c0 = lax.shift_right_logical(iota, 3)
        c1 = c0 + 2

        def chunk_body(c, carry):
            off = c * CH
            pltpu.sync_copy(src_hbm.at[pl.ds(off, CH)], src_v)
            pltpu.sync_copy(dst_hbm.at[pl.ds(off, CH)], dst_v)
            pltpu.sync_copy(ew_hbm.at[pl.ds(off, CH)], ew_v)
            pltpu.sync_copy(ex_hbm.at[pl.ds(off, CH)], ex_r)
            pltpu.async_copy(hp_hbm.at[src_v], hp_r, sem0).wait()
            pltpu.async_copy(den_hbm.at[dst_v], dn_r, sem1).wait()

            def w_body(j, carry2):
                e_idx = e_base4 + 4 * j
                col = hl_idx + hoff
                exv = plsc.load_gather(ex_r, [e_idx, col])
                dnv = plsc.load_gather(dn_r, [e_idx, col])
                ewv = plsc.load_gather(ew_v, [e_idx])
                wv = exv / (dnv + 1e-16) * ewv
                plsc.store_scatter(w_r, [e_idx * 4 + hl_idx], wv)
                return carry2

            lax.fori_loop(0, CH * (H // 2) // 16, w_body, 0)

            def expand_body(e, carry2):
                esp = jnp.full((16,), 4 * e, dtype=jnp.int32)
                wv0 = plsc.load_gather(w_r, [esp + c0])
                wv1 = plsc.load_gather(w_r, [esp + c1])
                msg_r[e, pl.ds(0, 16)] = hp_r[e, pl.ds(0, 16)] * wv0
                msg_r[e, pl.ds(16, 16)] = hp_r[e, pl.ds(16, 16)] * wv1
                return carry2

            lax.fori_loop(0, CH, expand_body, 0)
            pltpu.sync_copy(msg_r, out_sh.at[dst_v], add=True)
            return carry

        lax.fori_loop(base, base + cnt, chunk_body, 0)
        plsc.subcore_barrier()

        @pl.when(sid == 0)
        def _():
            pltpu.sync_copy(out_sh, out_hbm.at[cid])

    return _sc_pass_b


_sc_pass_b0 = _make_sc_pass_b(0)
_sc_pass_b1 = _make_sc_pass_b(H // 2)


@functools.partial(
    pl.kernel, mesh=_sc_mesh,
    compiler_params=pltpu.CompilerParams(needs_layout_passes=False, use_tc_tiling_on_sc=False),
    out_type=jax.ShapeDtypeStruct((E,), jnp.float32),
    scratch_types=[
        pltpu.VMEM((CH,), jnp.int32),
        pltpu.VMEM((CH,), jnp.int32),
        pltpu.VMEM((CH, D), jnp.float32),
        pltpu.VMEM((CH, D), jnp.float32),
        pltpu.VMEM((D,), jnp.float32),
        pltpu.VMEM((D,), jnp.float32),
        pltpu.VMEM((CH,), jnp.float32),
        pltpu.SemaphoreType.DMA,
        pltpu.SemaphoreType.DMA,
    ],
)
def _sc_lr(src_hbm, dst_hbm, p_hbm, q_hbm, b_hbm, w2_hbm,
           lr_hbm,
           src_v, dst_v, p_r, q_r, b_v, w2_v, out_v, sem0, sem1):
    cid = lax.axis_index("c")
    sid = lax.axis_index("s")
    base, cnt = _worker_bounds(cid, sid)
    pltpu.sync_copy(b_hbm, b_v)
    pltpu.sync_copy(w2_hbm, w2_v)

    def chunk_body(c, carry):
        off = c * CH
        pltpu.sync_copy(src_hbm.at[pl.ds(off, CH)], src_v)
        pltpu.sync_copy(dst_hbm.at[pl.ds(off, CH)], dst_v)
        pltpu.async_copy(p_hbm.at[src_v], p_r, sem0).wait()
        pltpu.async_copy(q_hbm.at[dst_v], q_r, sem1).wait()

        def edge_body(e, carry2):
            acc = jnp.zeros((16,), jnp.float32)
            for j in range(D // 16):
                s = pl.ds(16 * j, 16)
                t = jnp.maximum(p_r[e, s] + q_r[e, s] + b_v[s], 0.0)
                acc = acc + t * w2_v[s]
            s = jnp.sum(acc)
            plsc.store_scatter(out_v, [jnp.full((16,), e, jnp.int32)],
                               jnp.full((16,), s, jnp.float32),
                               mask=lax.iota(jnp.int32, 16) < 1)
            return carry2

        lax.fori_loop(0, CH, edge_body, 0)
        pltpu.sync_copy(out_v, lr_hbm.at[pl.ds(off, CH)])
        return carry

    lax.fori_loop(base, base + cnt, chunk_body, 0)


def _gat_edges(hp, ss, src, dst, edge_type, edge_weight, eb_l,
               zero16, zero32):
    ex, dpart = _sc_pass_a(src, dst, edge_type, ss, eb_l.reshape(-1), zero16)
    denom = dpart[0] + dpart[1]
    hpA = jnp.copy(hp[:, :D // 2])
    hpB = jnp.copy(hp[:, D // 2:])
    outA = _sc_pass_b0(src, dst, edge_weight, hpA, ex, denom, zero32)
    outB = _sc_pass_b1(src, dst, edge_weight, hpB, ex, denom, zero32)
    out = jnp.concatenate([outA[0] + outA[1], outB[0] + outB[1]], axis=1)
    return jax.nn.elu(out)


def _lr_head(P, Q, src, dst, blr1, Wlr2, blr2):
    return _sc_lr(src, dst, P, Q, blr1, Wlr2) + blr2


def kernel(x, edge_index, edge_type, edge_weight, W1, b1, W2, b2, W3, b3,
           Wg, a_src, a_dst, eb, Wjk, bjk, Wlr1, blr1, Wlr2, blr2, Wrec, brec):
    src = edge_index[0]
    dst = edge_index[1]

    eye = jnp.eye(H, dtype=jnp.float32)
    Asd = jnp.concatenate([
        (a_src[:, :, :, None] * eye[:, None, :]).reshape(L, D, H),
        (a_dst[:, :, :, None] * eye[:, None, :]).reshape(L, D, H),
    ], axis=-1)

    zero16 = jnp.zeros((N, 16), jnp.float32)
    zero32 = jnp.zeros((N, D // 2), jnp.float32)

    h = _encoder(x, W1, b1, W2, b2, W3, b3)
    outs = []
    for l in range(L):
        hp, ss = _gat_prep(h, Wg[l], Asd[l])
        h = _gat_edges(hp, ss, src, dst, edge_type, edge_weight, eb[l],
                       zero16, zero32)
        outs.append(h)

    P, Q, recon = _jk_head(outs[0], outs[1], outs[2], Wjk, bjk, Wlr1, Wrec, brec)
    lr_scores = _lr_head(P, Q, src, dst, blr1, Wlr2, blr2)
    return (lr_scores, recon)
